# baseline (speedup 1.0000x reference)
import functools

import jax
import jax.numpy as jnp
from jax import lax
from jax.experimental import pallas as pl
from jax.experimental.pallas import tpu as pltpu
from jax.experimental.pallas import tpu_sc as plsc

DIM_IN = 2048
DIM_OUT = 2048
NUM_SEGMENTS = 16
N_TOK = 4096

TM = 128
NB = N_TOK // TM
GRID = NB + NUM_SEGMENTS - 1

_NC = 2
_NS = 16
_NW = _NC * _NS
_ROWS_PER_W = N_TOK // _NW
_CH = 32
_NCHUNK = _ROWS_PER_W // _CH


def _row_gather(table, idx):
    ncols = table.shape[1]
    mesh = plsc.VectorSubcoreMesh(core_axis_name="c", subcore_axis_name="s")

    @functools.partial(
        pl.kernel,
        mesh=mesh,
        out_type=jax.ShapeDtypeStruct((N_TOK, ncols), jnp.float32),
        scratch_types=[
            pltpu.VMEM((_CH,), jnp.int32),
            pltpu.VMEM((_CH, ncols), jnp.float32),
            pltpu.SemaphoreType.DMA,
        ],
    )
    def k(table_hbm, idx_hbm, out_hbm, idx_v, rows_v, sem):
        wid = lax.axis_index("s") * _NC + lax.axis_index("c")
        base = wid * _ROWS_PER_W
        for c in range(_NCHUNK):
            off = base + c * _CH
            pltpu.sync_copy(idx_hbm.at[pl.ds(off, _CH)], idx_v)
            pltpu.async_copy(table_hbm.at[idx_v], rows_v, sem).wait()
            pltpu.sync_copy(rows_v, out_hbm.at[pl.ds(off, _CH)])

    return k(table, idx)


def _gmm_body(eid_ref, bid_ref, rs_ref, re_ref, first_ref, nxt_ref, slot_ref,
              x_ref, w_hbm, b_ref, o_ref, wbuf, sems):
    t = pl.program_id(0)
    rs = rs_ref[t]
    re = re_ref[t]
    slot = slot_ref[t]
    w3 = w_hbm

    @pl.when(t == 0)
    def _():
        pltpu.make_async_copy(w3.at[eid_ref[0]], wbuf.at[0], sems.at[0]).start()

    @pl.when(first_ref[t] == 1)
    def _():
        pltpu.make_async_copy(
            w3.at[eid_ref[t]], wbuf.at[slot], sems.at[slot]).wait()

        @pl.when(nxt_ref[t] >= 0)
        def _():
            pltpu.make_async_copy(
                w3.at[nxt_ref[t]], wbuf.at[1 - slot], sems.at[1 - slot]
            ).start()

    @pl.when(rs < re)
    def _():
        base = bid_ref[t] * TM
        rows = base + lax.broadcasted_iota(jnp.int32, (TM, 1), 0)
        mask = (rows >= rs) & (rows < re)
        acc = lax.dot_general(
            x_ref[...].astype(jnp.bfloat16),
            wbuf[slot],
            dimension_numbers=(((1,), (1,)), ((), ())),
            preferred_element_type=jnp.float32,
        )
        o_ref[...] = jnp.where(mask, acc + b_ref[0], o_ref[...])


def _grouped_gemm(x_sorted, weights, b3, eid, bid, rs, re, first, nxt, slot):
    return pl.pallas_call(
        _gmm_body,
        grid_spec=pltpu.PrefetchScalarGridSpec(
            num_scalar_prefetch=7,
            grid=(GRID,),
            in_specs=[
                pl.BlockSpec((TM, DIM_IN), lambda t, *s: (s[1][t], 0)),
                pl.BlockSpec(memory_space=pl.ANY),
                pl.BlockSpec((1, 1, DIM_OUT), lambda t, *s: (s[0][t], 0, 0)),
            ],
            out_specs=pl.BlockSpec((TM, DIM_OUT), lambda t, *s: (s[1][t], 0)),
            scratch_shapes=[
                pltpu.VMEM((2, DIM_OUT, DIM_IN), jnp.bfloat16),
                pltpu.SemaphoreType.DMA((2,)),
            ],
        ),
        out_shape=jax.ShapeDtypeStruct((N_TOK, DIM_OUT), jnp.float32),
    )(eid, bid, rs, re, first, nxt, slot, x_sorted, weights, b3)


def kernel(x, coords, weights, bias):
    xf = x.reshape(-1, DIM_IN)
    cf = coords.reshape(-1).astype(jnp.int32)

    perm = jnp.argsort(cf).astype(jnp.int32)
    inv_perm = jnp.zeros((N_TOK,), jnp.int32).at[perm].set(
        jnp.arange(N_TOK, dtype=jnp.int32))
    counts = jnp.zeros((NUM_SEGMENTS,), jnp.int32).at[cf].add(1)
    ends = jnp.cumsum(counts)
    starts = ends - counts
    first_blk = starts // TM
    tiles = jnp.where(counts > 0, (ends + TM - 1) // TM - first_blk, 0)
    inc = jnp.cumsum(tiles)
    t_idx = jnp.arange(GRID, dtype=jnp.int32)
    eid = jnp.minimum(
        jnp.searchsorted(inc, t_idx, side="right"), NUM_SEGMENTS - 1
    ).astype(jnp.int32)
    tile_off = inc - tiles
    valid = t_idx < inc[-1]
    bid = jnp.where(valid, first_blk[eid] + (t_idx - tile_off[eid]),
                    NB - 1).astype(jnp.int32)
    rs = jnp.where(valid, starts[eid], 0).astype(jnp.int32)
    re = jnp.where(valid, ends[eid], 0).astype(jnp.int32)

    e_prev = jnp.concatenate([jnp.full((1,), -1, jnp.int32), eid[:-1]])
    first = ((eid != e_prev) & valid).astype(jnp.int32)
    ordinal = jnp.cumsum(first) - 1
    slot = (ordinal % 2).astype(jnp.int32)
    n_distinct = jnp.sum(first)
    pos = jnp.where(first == 1, ordinal, GRID)
    order_e = jnp.full((GRID,), -1, jnp.int32).at[pos].set(eid, mode="drop")
    nxt = jnp.where(ordinal + 1 < n_distinct,
                    order_e[jnp.clip(ordinal + 1, 0, GRID - 1)],
                    -1).astype(jnp.int32)

    x_sorted = _row_gather(xf, perm)
    wb = weights.reshape(NUM_SEGMENTS, DIM_OUT, DIM_IN).astype(jnp.bfloat16)
    b3 = bias.reshape(NUM_SEGMENTS, 1, DIM_OUT)
    out_sorted = _grouped_gemm(x_sorted, wb, b3, eid, bid, rs, re,
                               first, nxt, slot)
    out = _row_gather(out_sorted, inv_perm)
    return out.reshape(*x.shape[:-1], DIM_OUT)


# speedup vs baseline: 1.2060x; 1.2060x over previous
import functools

import jax
import jax.numpy as jnp
from jax import lax
from jax.experimental import pallas as pl
from jax.experimental.pallas import tpu as pltpu
from jax.experimental.pallas import tpu_sc as plsc

DIM_IN = 2048
DIM_OUT = 2048
NUM_SEGMENTS = 16
N_TOK = 4096

TM = 256
NB = N_TOK // TM
GRID = NB + NUM_SEGMENTS - 1

_NC = 2
_NS = 16
_NW = _NC * _NS
_ROWS_PER_W = N_TOK // _NW
_CH = 32
_NCHUNK = _ROWS_PER_W // _CH


def _row_gather(table, idx):
    ncols = table.shape[1]
    mesh = plsc.VectorSubcoreMesh(core_axis_name="c", subcore_axis_name="s")

    @functools.partial(
        pl.kernel,
        mesh=mesh,
        out_type=jax.ShapeDtypeStruct((N_TOK, ncols), jnp.float32),
        scratch_types=[
            pltpu.VMEM((_CH,), jnp.int32),
            pltpu.VMEM((_CH, ncols), jnp.float32),
            pltpu.SemaphoreType.DMA,
        ],
    )
    def k(table_hbm, idx_hbm, out_hbm, idx_v, rows_v, sem):
        wid = lax.axis_index("s") * _NC + lax.axis_index("c")
        base = wid * _ROWS_PER_W
        for c in range(_NCHUNK):
            off = base + c * _CH
            pltpu.sync_copy(idx_hbm.at[pl.ds(off, _CH)], idx_v)
            pltpu.async_copy(table_hbm.at[idx_v], rows_v, sem).wait()
            pltpu.sync_copy(rows_v, out_hbm.at[pl.ds(off, _CH)])

    return k(table, idx)


def _gmm_body(eid_ref, bid_ref, rs_ref, re_ref, first_ref, nxt_ref, slot_ref,
              x_ref, w_hbm, b_ref, o_ref, wbuf, wb16, sems):
    t = pl.program_id(0)
    rs = rs_ref[t]
    re = re_ref[t]
    slot = slot_ref[t]
    w3 = w_hbm

    @pl.when(t == 0)
    def _():
        pltpu.make_async_copy(w3.at[eid_ref[0]], wbuf.at[0], sems.at[0]).start()

    @pl.when(first_ref[t] == 1)
    def _():
        pltpu.make_async_copy(
            w3.at[eid_ref[t]], wbuf.at[slot], sems.at[slot]).wait()

        @pl.when(nxt_ref[t] >= 0)
        def _():
            pltpu.make_async_copy(
                w3.at[nxt_ref[t]], wbuf.at[1 - slot], sems.at[1 - slot]
            ).start()

        wb16[...] = wbuf[slot].astype(jnp.bfloat16)

    @pl.when(rs < re)
    def _():
        base = bid_ref[t] * TM
        rows = base + lax.broadcasted_iota(jnp.int32, (TM, 1), 0)
        mask = (rows >= rs) & (rows < re)
        acc = lax.dot_general(
            x_ref[...].astype(jnp.bfloat16),
            wb16[...],
            dimension_numbers=(((1,), (1,)), ((), ())),
            preferred_element_type=jnp.float32,
        )
        o_ref[...] = jnp.where(mask, acc + b_ref[0], o_ref[...])


def _grouped_gemm(x_sorted, weights, b3, eid, bid, rs, re, first, nxt, slot):
    return pl.pallas_call(
        _gmm_body,
        grid_spec=pltpu.PrefetchScalarGridSpec(
            num_scalar_prefetch=7,
            grid=(GRID,),
            in_specs=[
                pl.BlockSpec((TM, DIM_IN), lambda t, *s: (s[1][t], 0)),
                pl.BlockSpec(memory_space=pl.ANY),
                pl.BlockSpec((1, 1, DIM_OUT), lambda t, *s: (s[0][t], 0, 0)),
            ],
            out_specs=pl.BlockSpec((TM, DIM_OUT), lambda t, *s: (s[1][t], 0)),
            scratch_shapes=[
                pltpu.VMEM((2, DIM_OUT, DIM_IN), jnp.float32),
                pltpu.VMEM((DIM_OUT, DIM_IN), jnp.bfloat16),
                pltpu.SemaphoreType.DMA((2,)),
            ],
        ),
        out_shape=jax.ShapeDtypeStruct((N_TOK, DIM_OUT), jnp.float32),
    )(eid, bid, rs, re, first, nxt, slot, x_sorted, weights, b3)


def kernel(x, coords, weights, bias):
    xf = x.reshape(-1, DIM_IN)
    cf = coords.reshape(-1).astype(jnp.int32)

    perm = jnp.argsort(cf).astype(jnp.int32)
    inv_perm = jnp.zeros((N_TOK,), jnp.int32).at[perm].set(
        jnp.arange(N_TOK, dtype=jnp.int32))
    counts = jnp.zeros((NUM_SEGMENTS,), jnp.int32).at[cf].add(1)
    ends = jnp.cumsum(counts)
    starts = ends - counts
    first_blk = starts // TM
    tiles = jnp.where(counts > 0, (ends + TM - 1) // TM - first_blk, 0)
    inc = jnp.cumsum(tiles)
    t_idx = jnp.arange(GRID, dtype=jnp.int32)
    eid = jnp.minimum(
        jnp.searchsorted(inc, t_idx, side="right"), NUM_SEGMENTS - 1
    ).astype(jnp.int32)
    tile_off = inc - tiles
    valid = t_idx < inc[-1]
    bid = jnp.where(valid, first_blk[eid] + (t_idx - tile_off[eid]),
                    NB - 1).astype(jnp.int32)
    rs = jnp.where(valid, starts[eid], 0).astype(jnp.int32)
    re = jnp.where(valid, ends[eid], 0).astype(jnp.int32)

    e_prev = jnp.concatenate([jnp.full((1,), -1, jnp.int32), eid[:-1]])
    first = ((eid != e_prev) & valid).astype(jnp.int32)
    ordinal = jnp.cumsum(first) - 1
    slot = (ordinal % 2).astype(jnp.int32)
    n_distinct = jnp.sum(first)
    pos = jnp.where(first == 1, ordinal, GRID)
    order_e = jnp.full((GRID,), -1, jnp.int32).at[pos].set(eid, mode="drop")
    nxt = jnp.where(ordinal + 1 < n_distinct,
                    order_e[jnp.clip(ordinal + 1, 0, GRID - 1)],
                    -1).astype(jnp.int32)

    x_sorted = _row_gather(xf, perm)
    w3 = weights.reshape(NUM_SEGMENTS, DIM_OUT, DIM_IN)
    b3 = bias.reshape(NUM_SEGMENTS, 1, DIM_OUT)
    out_sorted = _grouped_gemm(x_sorted, w3, b3, eid, bid, rs, re,
                               first, nxt, slot)
    out = _row_gather(out_sorted, inv_perm)
    return out.reshape(*x.shape[:-1], DIM_OUT)
